# baseline (speedup 1.0000x reference)
import functools

import jax
import jax.numpy as jnp
from jax import lax
from jax.experimental import pallas as pl
from jax.experimental.pallas import tpu as pltpu
from jax.experimental.pallas import tpu_sc as plsc

_N = 10000
_E = 320000
_NP = 10240
_EP = 331776
_ROWS_PER_TILE = _NP // 16

_mesh = plsc.VectorSubcoreMesh(core_axis_name="c", subcore_axis_name="s")


def _sc_layer(D, H, ROWLEN, K, EP, CHUNK_B):
    CH = D // H
    NB = EP // (32 * K)
    NCHUNK = NB // CHUNK_B
    RING = 4

    @functools.partial(
        pl.kernel,
        mesh=_mesh,
        compiler_params=pltpu.CompilerParams(
            needs_layout_passes=False, use_tc_tiling_on_sc=False),
        out_type=jax.ShapeDtypeStruct((2, _NP, ROWLEN), jnp.float32),
        scratch_types=[
            pltpu.VMEM_SHARED((_NP, ROWLEN), jnp.float32),
            pltpu.VMEM((CHUNK_B, 2 * K), jnp.int32),
            pltpu.VMEM((CHUNK_B, K), jnp.int32),
            pltpu.VMEM((RING, 2 * K, D), jnp.float32),
            pltpu.VMEM((2, K, ROWLEN), jnp.float32),
            pltpu.VMEM((D,), jnp.float32),
            pltpu.SemaphoreType.DMA,
            pltpu.SemaphoreType.DMA,
            pltpu.SemaphoreType.DMA,
            pltpu.SemaphoreType.DMA,
            pltpu.SemaphoreType.DMA,
            pltpu.SemaphoreType.DMA,
        ],
    )
    def body(t_h, gidx_h, dst2_h, att_h, out_h,
             acc_s, gidxb, dstb, xlrb, wxlb, att_v, g0, g1, g2, g3, s0, s1):
        c = lax.axis_index("c")
        s = lax.axis_index("s")
        wid = s * 2 + c
        wbase = wid * NB

        pltpu.sync_copy(att_h, att_v)

        zv = jnp.zeros((16,), jnp.float32)

        def zb(r, carry):
            for b in range(2):
                for cc in range(ROWLEN // 16):
                    wxlb[b, r, pl.ds(cc * 16, 16)] = zv
            return carry

        lax.fori_loop(0, K, zb, 0)
        for t in range(_ROWS_PER_TILE // K):
            pltpu.sync_copy(wxlb.at[0],
                            acc_s.at[pl.ds(s * _ROWS_PER_TILE + t * K, K)])
        plsc.subcore_barrier()

        gsems = (g0, g1, g2, g3)
        ssems = (s0, s1)

        def drain_scatter(b):
            pltpu.make_async_copy(out_h.at[0, pl.ds(0, K)], wxlb.at[b],
                                  ssems[b]).wait()

        def load_chunk(cc):
            pltpu.sync_copy(gidx_h.at[pl.ds(wbase + cc * CHUNK_B, CHUNK_B)],
                            gidxb)
            pltpu.sync_copy(dst2_h.at[pl.ds(wbase + cc * CHUNK_B, CHUNK_B)],
                            dstb)

        def issue_gather(jl, b):
            pltpu.async_copy(t_h.at[gidxb.at[jl]], xlrb.at[b], gsems[b])

        load_chunk(0)
        for b in range(RING):
            issue_gather(b, b)

        def chunk(cc, carry):
            @pl.when(cc > 0)
            def _():
                drain_scatter(0)
                drain_scatter(1)
                load_chunk(cc)
                for b in range(RING):
                    issue_gather(b, b)

            def group(jj, carry2):
                for b in range(RING):
                    jl = jj * RING + b
                    pltpu.make_async_copy(t_h.at[gidxb.at[jl]], xlrb.at[b],
                                          gsems[b]).wait()
                    @pl.when(jl >= 2)
                    def _():
                        drain_scatter(b % 2)

                    def eg_body(eg, carry3):
                        rowv = eg * 16 + lax.iota(jnp.int32, 16)
                        xb = xlrb.at[b]
                        wb = wxlb.at[b % 2]
                        for h in range(H):
                            acc = jnp.zeros((16,), jnp.float32)
                            att_rows = [att_v[pl.ds(h * CH + k * 16, 16)]
                                        for k in range(CH // 16)]
                            saved = []
                            for c0 in range(CH):
                                ch = h * CH + c0
                                colv = jnp.full((16,), ch, jnp.int32)
                                xlv = plsc.load_gather(xb, [rowv, colv])
                                xrv = plsc.load_gather(xb, [rowv + K, colv])
                                sv = xlv + xrv
                                lv = jnp.maximum(sv, sv * 0.2)
                                acc = acc + lv * att_rows[c0 // 16][c0 % 16]
                                if CH <= 16:
                                    saved.append((colv, xlv))
                            wv = jnp.exp(acc)
                            if CH <= 16:
                                for colv, xlv in saved:
                                    plsc.store_scatter(wb, [rowv, colv],
                                                       xlv * wv)
                            else:
                                for c0 in range(CH):
                                    ch = h * CH + c0
                                    colv = jnp.full((16,), ch, jnp.int32)
                                    xlv = plsc.load_gather(xb, [rowv, colv])
                                    plsc.store_scatter(wb, [rowv, colv],
                                                       xlv * wv)
                            plsc.store_scatter(
                                wb, [rowv, jnp.full((16,), D + h, jnp.int32)],
                                wv)
                        return carry3

                    lax.fori_loop(0, K // 16, eg_body, 0)
                    pltpu.async_copy(wxlb.at[b % 2], acc_s.at[dstb.at[jl]],
                                     ssems[b % 2], add=True)
                    @pl.when(jl + RING < CHUNK_B)
                    def _():
                        issue_gather(jl + RING, b)
                return carry2

            lax.fori_loop(0, CHUNK_B // RING, group, 0)
            return carry

        lax.fori_loop(0, NCHUNK, chunk, 0)
        drain_scatter(0)
        drain_scatter(1)
        plsc.subcore_barrier()
        pltpu.sync_copy(acc_s.at[pl.ds(s * _ROWS_PER_TILE, _ROWS_PER_TILE)],
                        out_h.at[c, pl.ds(s * _ROWS_PER_TILE, _ROWS_PER_TILE)])

    return body


_K1 = 16
_EP1 = 331776
_K2 = 64
_EP2 = 335872
_sc_layer1 = _sc_layer(128, 8, 144, _K1, _EP1, 108)
_sc_layer2 = _sc_layer(48, 1, 64, _K2, _EP2, 164)


def _tc_mm2(x, Wa, Wb):
    n, f = x.shape
    d = Wa.shape[1]
    B = 1024

    def body(x_r, wa_r, wb_r, oa_r, ob_r):
        xb = x_r[...]
        oa_r[...] = jnp.dot(xb, wa_r[...], preferred_element_type=jnp.float32)
        ob_r[...] = jnp.dot(xb, wb_r[...], preferred_element_type=jnp.float32)

    return pl.pallas_call(
        body,
        grid=(n // B,),
        in_specs=[
            pl.BlockSpec((B, f), lambda i: (i, 0)),
            pl.BlockSpec((f, d), lambda i: (0, 0)),
            pl.BlockSpec((f, d), lambda i: (0, 0)),
        ],
        out_specs=[
            pl.BlockSpec((B, d), lambda i: (i, 0)),
            pl.BlockSpec((B, d), lambda i: (i, 0)),
        ],
        out_shape=[
            jax.ShapeDtypeStruct((n, d), jnp.float32),
            jax.ShapeDtypeStruct((n, d), jnp.float32),
        ],
    )(x, Wa, Wb)


def _tc_combine1_mm(p0, p1, b1, Wl2, Wr2):
    n = p0.shape[0]
    B = 1024
    d2 = Wl2.shape[1]

    def body(p0_r, p1_r, b1_r, wl_r, wr_r, oa_r, ob_r):
        p = p0_r[...] + p1_r[...]
        num = p[:, :128]
        den = p[:, 128:136]
        denb = jnp.broadcast_to(den.reshape(B, 8, 1), (B, 8, 16)).reshape(B, 128)
        h = jnp.maximum(num / (denb + 1e-16) + b1_r[...], 0.0)
        oa_r[...] = jnp.dot(h, wl_r[...], preferred_element_type=jnp.float32)
        ob_r[...] = jnp.dot(h, wr_r[...], preferred_element_type=jnp.float32)

    return pl.pallas_call(
        body,
        grid=(n // B,),
        in_specs=[
            pl.BlockSpec((B, 144), lambda i: (i, 0)),
            pl.BlockSpec((B, 144), lambda i: (i, 0)),
            pl.BlockSpec((1, 128), lambda i: (0, 0)),
            pl.BlockSpec((128, d2), lambda i: (0, 0)),
            pl.BlockSpec((128, d2), lambda i: (0, 0)),
        ],
        out_specs=[
            pl.BlockSpec((B, d2), lambda i: (i, 0)),
            pl.BlockSpec((B, d2), lambda i: (i, 0)),
        ],
        out_shape=[
            jax.ShapeDtypeStruct((n, d2), jnp.float32),
            jax.ShapeDtypeStruct((n, d2), jnp.float32),
        ],
    )(p0, p1, b1, Wl2, Wr2)


def _tc_combine2(q0, q1, b2):
    n = q0.shape[0]
    B = 1024

    def body(q0_r, q1_r, b2_r, o_r):
        q = q0_r[...] + q1_r[...]
        den = jnp.broadcast_to(q[:, 48:49], (B, 64))
        o_r[...] = q / (den + 1e-16) + b2_r[...]

    return pl.pallas_call(
        body,
        grid=(n // B,),
        in_specs=[
            pl.BlockSpec((B, 64), lambda i: (i, 0)),
            pl.BlockSpec((B, 64), lambda i: (i, 0)),
            pl.BlockSpec((1, 64), lambda i: (0, 0)),
        ],
        out_specs=pl.BlockSpec((B, 64), lambda i: (i, 0)),
        out_shape=jax.ShapeDtypeStruct((n, 64), jnp.float32),
    )(q0, q1, b2)


def _pad_edges(adj_t, ep):
    loops = jnp.arange(_N, dtype=jnp.int32)
    padi = jnp.full((ep - _E - _N,), _N, dtype=jnp.int32)
    src = jnp.concatenate([adj_t[0].astype(jnp.int32), loops, padi])
    dst = jnp.concatenate([adj_t[1].astype(jnp.int32), loops, padi])
    return src, dst


def kernel(x, adj_t, Wl1, Wr1, att1, b1, Wl2, Wr2, att2, b2):
    src1, dst1 = _pad_edges(adj_t, _EP1)
    gidx1 = jnp.concatenate(
        [src1.reshape(-1, _K1), dst1.reshape(-1, _K1) + _NP], axis=1)
    dstb1 = dst1.reshape(-1, _K1)
    src2, dst2 = _pad_edges(adj_t, _EP2)
    gidx2 = jnp.concatenate(
        [src2.reshape(-1, _K2), dst2.reshape(-1, _K2) + _NP], axis=1)
    dstb2 = dst2.reshape(-1, _K2)

    xp = jnp.pad(x, ((0, _NP - _N), (0, 0)))
    xl1, xr1 = _tc_mm2(xp, Wl1, Wr1)
    t1 = jnp.concatenate([xl1, xr1])

    att1f = att1.reshape(128)
    parts1 = _sc_layer1(t1, gidx1, dstb1, att1f)

    Wl2p = jnp.pad(Wl2, ((0, 0), (0, 8)))
    Wr2p = jnp.pad(Wr2, ((0, 0), (0, 8)))
    hl2, hr2 = _tc_combine1_mm(parts1[0], parts1[1], b1.reshape(1, 128),
                               Wl2p, Wr2p)
    t2 = jnp.concatenate([hl2, hr2])

    att2f = jnp.pad(att2.reshape(40), (0, 8))
    parts2 = _sc_layer2(t2, gidx2, dstb2, att2f)

    b2p = jnp.pad(b2, (0, 24)).reshape(1, 64)
    outp = _tc_combine2(parts2[0], parts2[1], b2p)
    return outp[:_N, :40]


# speedup vs baseline: 1.2224x; 1.2224x over previous
import functools

import jax
import jax.numpy as jnp
from jax import lax
from jax.experimental import pallas as pl
from jax.experimental.pallas import tpu as pltpu
from jax.experimental.pallas import tpu_sc as plsc

_N = 10000
_E = 320000
_NP = 10240
_EP = 331776
_ROWS_PER_TILE = _NP // 16

_mesh = plsc.VectorSubcoreMesh(core_axis_name="c", subcore_axis_name="s")


def _sc_layer(D, H, ROWLEN, K, EP, CHUNK_B, RING=2, TBL_SPMEM=False):
    CH = D // H
    NB = EP // (32 * K)
    NCHUNK = NB // CHUNK_B
    TROWS = (2 * _NP) // 16

    @functools.partial(
        pl.kernel,
        mesh=_mesh,
        compiler_params=pltpu.CompilerParams(
            needs_layout_passes=False, use_tc_tiling_on_sc=False),
        out_type=jax.ShapeDtypeStruct((2, _NP, ROWLEN), jnp.float32),
        scratch_types=(
            [pltpu.VMEM_SHARED((_NP, ROWLEN), jnp.float32)]
            + ([pltpu.VMEM_SHARED((2 * _NP, D), jnp.float32)]
               if TBL_SPMEM else [])
            + [
                pltpu.VMEM((CHUNK_B, 2 * K), jnp.int32),
                pltpu.VMEM((CHUNK_B, K), jnp.int32),
                pltpu.VMEM((RING, 2 * K, D), jnp.float32),
                pltpu.VMEM((2, K, ROWLEN), jnp.float32),
                pltpu.VMEM((D,), jnp.float32),
            ]
            + [pltpu.SemaphoreType.DMA] * (RING + 2)
        ),
    )
    def body(t_h, gidx_h, dst2_h, att_h, out_h, acc_s, *rest):
        if TBL_SPMEM:
            tbl_s = rest[0]
            rest = rest[1:]
        gidxb, dstb, xlrb, wxlb, att_v = rest[:5]
        gsems = rest[5:5 + RING]
        ssems = rest[5 + RING:5 + RING + 2]
        c = lax.axis_index("c")
        s = lax.axis_index("s")
        wid = s * 2 + c
        wbase = wid * NB

        pltpu.sync_copy(att_h, att_v)

        zv = jnp.zeros((16,), jnp.float32)

        def zb(r, carry):
            for b in range(2):
                for cc in range(ROWLEN // 16):
                    wxlb[b, r, pl.ds(cc * 16, 16)] = zv
            return carry

        lax.fori_loop(0, K, zb, 0)
        for t in range(_ROWS_PER_TILE // K):
            pltpu.sync_copy(wxlb.at[0],
                            acc_s.at[pl.ds(s * _ROWS_PER_TILE + t * K, K)])
        if TBL_SPMEM:
            pltpu.sync_copy(t_h.at[pl.ds(s * TROWS, TROWS)],
                            tbl_s.at[pl.ds(s * TROWS, TROWS)])
        plsc.subcore_barrier()

        t_src = tbl_s if TBL_SPMEM else t_h

        def drain_scatter(b):
            pltpu.make_async_copy(out_h.at[0, pl.ds(0, K)], wxlb.at[b],
                                  ssems[b]).wait()

        def load_chunk(cc):
            pltpu.sync_copy(gidx_h.at[pl.ds(wbase + cc * CHUNK_B, CHUNK_B)],
                            gidxb)
            pltpu.sync_copy(dst2_h.at[pl.ds(wbase + cc * CHUNK_B, CHUNK_B)],
                            dstb)

        def issue_gather(jl, b):
            pltpu.async_copy(t_src.at[gidxb.at[jl]], xlrb.at[b], gsems[b])

        load_chunk(0)
        for b in range(RING):
            issue_gather(b, b)

        def chunk(cc, carry):
            @pl.when(cc > 0)
            def _():
                drain_scatter(0)
                drain_scatter(1)
                load_chunk(cc)
                for b in range(RING):
                    issue_gather(b, b)

            def group(jj, carry2):
                for b in range(RING):
                    jl = jj * RING + b
                    pltpu.make_async_copy(t_src.at[gidxb.at[jl]],
                                          xlrb.at[b], gsems[b]).wait()
                    @pl.when(jl >= 2)
                    def _():
                        drain_scatter(b % 2)

                    def eg_body(eg, carry3):
                        rowv = eg * 16 + lax.iota(jnp.int32, 16)
                        xb = xlrb.at[b]
                        wb = wxlb.at[b % 2]
                        for h in range(H):
                            acc = jnp.zeros((16,), jnp.float32)
                            att_rows = [att_v[pl.ds(h * CH + k * 16, 16)]
                                        for k in range(CH // 16)]
                            saved = []
                            for c0 in range(CH):
                                ch = h * CH + c0
                                colv = jnp.full((16,), ch, jnp.int32)
                                xlv = plsc.load_gather(xb, [rowv, colv])
                                xrv = plsc.load_gather(xb, [rowv + K, colv])
                                sv = xlv + xrv
                                lv = jnp.maximum(sv, sv * 0.2)
                                acc = acc + lv * att_rows[c0 // 16][c0 % 16]
                                if CH <= 16:
                                    saved.append((colv, xlv))
                            wv = jnp.exp(acc)
                            if CH <= 16:
                                for colv, xlv in saved:
                                    plsc.store_scatter(wb, [rowv, colv],
                                                       xlv * wv)
                            else:
                                for c0 in range(CH):
                                    ch = h * CH + c0
                                    colv = jnp.full((16,), ch, jnp.int32)
                                    xlv = plsc.load_gather(xb, [rowv, colv])
                                    plsc.store_scatter(wb, [rowv, colv],
                                                       xlv * wv)
                            plsc.store_scatter(
                                wb, [rowv, jnp.full((16,), D + h, jnp.int32)],
                                wv)
                        return carry3

                    lax.fori_loop(0, K // 16, eg_body, 0)
                    pltpu.async_copy(wxlb.at[b % 2], acc_s.at[dstb.at[jl]],
                                     ssems[b % 2], add=True)
                    @pl.when(jl + RING < CHUNK_B)
                    def _():
                        issue_gather(jl + RING, b)
                return carry2

            lax.fori_loop(0, CHUNK_B // RING, group, 0)
            return carry

        lax.fori_loop(0, NCHUNK, chunk, 0)
        drain_scatter(0)
        drain_scatter(1)
        plsc.subcore_barrier()
        pltpu.sync_copy(acc_s.at[pl.ds(s * _ROWS_PER_TILE, _ROWS_PER_TILE)],
                        out_h.at[c, pl.ds(s * _ROWS_PER_TILE, _ROWS_PER_TILE)])

    return body


_K1 = 32
_EP1 = 331776
_K2 = 32
_EP2 = 331776
_sc_layer1 = _sc_layer(128, 8, 144, _K1, _EP1, 54, RING=2)
_sc_layer2 = _sc_layer(48, 1, 64, _K2, _EP2, 108, RING=2, TBL_SPMEM=True)


def _tc_mm2(x, Wa, Wb):
    n, f = x.shape
    d = Wa.shape[1]
    B = 1024

    def body(x_r, wa_r, wb_r, oa_r, ob_r):
        xb = x_r[...]
        oa_r[...] = jnp.dot(xb, wa_r[...], preferred_element_type=jnp.float32)
        ob_r[...] = jnp.dot(xb, wb_r[...], preferred_element_type=jnp.float32)

    return pl.pallas_call(
        body,
        grid=(n // B,),
        in_specs=[
            pl.BlockSpec((B, f), lambda i: (i, 0)),
            pl.BlockSpec((f, d), lambda i: (0, 0)),
            pl.BlockSpec((f, d), lambda i: (0, 0)),
        ],
        out_specs=[
            pl.BlockSpec((B, d), lambda i: (i, 0)),
            pl.BlockSpec((B, d), lambda i: (i, 0)),
        ],
        out_shape=[
            jax.ShapeDtypeStruct((n, d), jnp.float32),
            jax.ShapeDtypeStruct((n, d), jnp.float32),
        ],
    )(x, Wa, Wb)


def _tc_combine1_mm(p0, p1, b1, Wl2, Wr2):
    n = p0.shape[0]
    B = 1024
    d2 = Wl2.shape[1]

    def body(p0_r, p1_r, b1_r, wl_r, wr_r, oa_r, ob_r):
        p = p0_r[...] + p1_r[...]
        num = p[:, :128]
        den = p[:, 128:136]
        denb = jnp.broadcast_to(den.reshape(B, 8, 1), (B, 8, 16)).reshape(B, 128)
        h = jnp.maximum(num / (denb + 1e-16) + b1_r[...], 0.0)
        oa_r[...] = jnp.dot(h, wl_r[...], preferred_element_type=jnp.float32)
        ob_r[...] = jnp.dot(h, wr_r[...], preferred_element_type=jnp.float32)

    return pl.pallas_call(
        body,
        grid=(n // B,),
        in_specs=[
            pl.BlockSpec((B, 144), lambda i: (i, 0)),
            pl.BlockSpec((B, 144), lambda i: (i, 0)),
            pl.BlockSpec((1, 128), lambda i: (0, 0)),
            pl.BlockSpec((128, d2), lambda i: (0, 0)),
            pl.BlockSpec((128, d2), lambda i: (0, 0)),
        ],
        out_specs=[
            pl.BlockSpec((B, d2), lambda i: (i, 0)),
            pl.BlockSpec((B, d2), lambda i: (i, 0)),
        ],
        out_shape=[
            jax.ShapeDtypeStruct((n, d2), jnp.float32),
            jax.ShapeDtypeStruct((n, d2), jnp.float32),
        ],
    )(p0, p1, b1, Wl2, Wr2)


def _tc_combine2(q0, q1, b2):
    n = q0.shape[0]
    B = 1024

    def body(q0_r, q1_r, b2_r, o_r):
        q = q0_r[...] + q1_r[...]
        den = jnp.broadcast_to(q[:, 48:49], (B, 64))
        o_r[...] = q / (den + 1e-16) + b2_r[...]

    return pl.pallas_call(
        body,
        grid=(n // B,),
        in_specs=[
            pl.BlockSpec((B, 64), lambda i: (i, 0)),
            pl.BlockSpec((B, 64), lambda i: (i, 0)),
            pl.BlockSpec((1, 64), lambda i: (0, 0)),
        ],
        out_specs=pl.BlockSpec((B, 64), lambda i: (i, 0)),
        out_shape=jax.ShapeDtypeStruct((n, 64), jnp.float32),
    )(q0, q1, b2)


def _pad_edges(adj_t, ep):
    loops = jnp.arange(_N, dtype=jnp.int32)
    padi = jnp.full((ep - _E - _N,), _N, dtype=jnp.int32)
    src = jnp.concatenate([adj_t[0].astype(jnp.int32), loops, padi])
    dst = jnp.concatenate([adj_t[1].astype(jnp.int32), loops, padi])
    return src, dst


def kernel(x, adj_t, Wl1, Wr1, att1, b1, Wl2, Wr2, att2, b2):
    src1, dst1 = _pad_edges(adj_t, _EP1)
    gidx1 = jnp.concatenate(
        [src1.reshape(-1, _K1), dst1.reshape(-1, _K1) + _NP], axis=1)
    dstb1 = dst1.reshape(-1, _K1)
    src2, dst2 = _pad_edges(adj_t, _EP2)
    gidx2 = jnp.concatenate(
        [src2.reshape(-1, _K2), dst2.reshape(-1, _K2) + _NP], axis=1)
    dstb2 = dst2.reshape(-1, _K2)

    xp = jnp.pad(x, ((0, _NP - _N), (0, 0)))
    xl1, xr1 = _tc_mm2(xp, Wl1, Wr1)
    t1 = jnp.concatenate([xl1, xr1])

    att1f = att1.reshape(128)
    parts1 = _sc_layer1(t1, gidx1, dstb1, att1f)

    Wl2p = jnp.pad(Wl2, ((0, 0), (0, 8)))
    Wr2p = jnp.pad(Wr2, ((0, 0), (0, 8)))
    hl2, hr2 = _tc_combine1_mm(parts1[0], parts1[1], b1.reshape(1, 128),
                               Wl2p, Wr2p)
    t2 = jnp.concatenate([hl2, hr2])

    att2f = jnp.pad(att2.reshape(40), (0, 8))
    parts2 = _sc_layer2(t2, gidx2, dstb2, att2f)

    b2p = jnp.pad(b2, (0, 24)).reshape(1, 64)
    outp = _tc_combine2(parts2[0], parts2[1], b2p)
    return outp[:_N, :40]


# speedup vs baseline: 1.3671x; 1.1184x over previous
import functools

import jax
import jax.numpy as jnp
from jax import lax
from jax.experimental import pallas as pl
from jax.experimental.pallas import tpu as pltpu
from jax.experimental.pallas import tpu_sc as plsc

_N = 10000
_E = 320000
_NP = 10240
_EP = 331776
_ROWS_PER_TILE = _NP // 16

_mesh = plsc.VectorSubcoreMesh(core_axis_name="c", subcore_axis_name="s")


def _sc_layer(D, H, ACT, ROWLEN, K, EP, CHUNK_B, RING=2, TBL_SPMEM=False):
    CH = D // H
    DS = H * ACT
    NB = EP // (32 * K)
    NCHUNK = NB // CHUNK_B
    TROWS = (2 * _NP) // 16

    @functools.partial(
        pl.kernel,
        mesh=_mesh,
        compiler_params=pltpu.CompilerParams(
            needs_layout_passes=False, use_tc_tiling_on_sc=False),
        out_type=jax.ShapeDtypeStruct((2, _NP, ROWLEN), jnp.float32),
        scratch_types=(
            [pltpu.VMEM_SHARED((_NP, ROWLEN), jnp.float32)]
            + ([pltpu.VMEM_SHARED((2 * _NP, D), jnp.float32)]
               if TBL_SPMEM else [])
            + [
                pltpu.VMEM((CHUNK_B, 2 * K), jnp.int32),
                pltpu.VMEM((CHUNK_B, K), jnp.int32),
                pltpu.VMEM((RING, 2 * K, D), jnp.float32),
                pltpu.VMEM((2, K, ROWLEN), jnp.float32),
                pltpu.VMEM((D,), jnp.float32),
            ]
            + [pltpu.SemaphoreType.DMA] * (RING + 2)
        ),
    )
    def body(t_h, gidx_h, dst2_h, att_h, out_h, acc_s, *rest):
        if TBL_SPMEM:
            tbl_s = rest[0]
            rest = rest[1:]
        gidxb, dstb, xlrb, wxlb, att_v = rest[:5]
        gsems = rest[5:5 + RING]
        ssems = rest[5 + RING:5 + RING + 2]
        c = lax.axis_index("c")
        s = lax.axis_index("s")
        wid = s * 2 + c
        wbase = wid * NB

        pltpu.sync_copy(att_h, att_v)

        zv = jnp.zeros((16,), jnp.float32)

        def zb(r, carry):
            for b in range(2):
                for cc in range(ROWLEN // 16):
                    wxlb[b, r, pl.ds(cc * 16, 16)] = zv
            return carry

        lax.fori_loop(0, K, zb, 0)
        for t in range(_ROWS_PER_TILE // K):
            pltpu.sync_copy(wxlb.at[0],
                            acc_s.at[pl.ds(s * _ROWS_PER_TILE + t * K, K)])
        if TBL_SPMEM:
            pltpu.sync_copy(t_h.at[pl.ds(s * TROWS, TROWS)],
                            tbl_s.at[pl.ds(s * TROWS, TROWS)])
        plsc.subcore_barrier()

        t_src = tbl_s if TBL_SPMEM else t_h

        def drain_scatter(b):
            pltpu.make_async_copy(out_h.at[0, pl.ds(0, K)], wxlb.at[b],
                                  ssems[b]).wait()

        def load_chunk(cc):
            pltpu.sync_copy(gidx_h.at[pl.ds(wbase + cc * CHUNK_B, CHUNK_B)],
                            gidxb)
            pltpu.sync_copy(dst2_h.at[pl.ds(wbase + cc * CHUNK_B, CHUNK_B)],
                            dstb)

        def issue_gather(jl, b):
            pltpu.async_copy(t_src.at[gidxb.at[jl]], xlrb.at[b], gsems[b])

        load_chunk(0)
        for b in range(RING):
            issue_gather(b, b)

        def chunk(cc, carry):
            @pl.when(cc > 0)
            def _():
                drain_scatter(0)
                drain_scatter(1)
                load_chunk(cc)
                for b in range(RING):
                    issue_gather(b, b)

            def group(jj, carry2):
                for b in range(RING):
                    jl = jj * RING + b
                    pltpu.make_async_copy(t_src.at[gidxb.at[jl]],
                                          xlrb.at[b], gsems[b]).wait()
                    @pl.when(jl >= 2)
                    def _():
                        drain_scatter(b % 2)

                    def eg_body(eg, carry3):
                        rowv = eg * 16 + lax.iota(jnp.int32, 16)
                        xb = xlrb.at[b]
                        wb = wxlb.at[b % 2]
                        for h in range(H):
                            acc = jnp.zeros((16,), jnp.float32)
                            att_rows = [att_v[pl.ds(h * CH + k * 16, 16)]
                                        for k in range(-(-ACT // 16))]
                            saved = []
                            for c0 in range(ACT):
                                cg = jnp.full((16,), h * CH + c0, jnp.int32)
                                xlv = plsc.load_gather(xb, [rowv, cg])
                                xrv = plsc.load_gather(xb, [rowv + K, cg])
                                sv = xlv + xrv
                                lv = jnp.maximum(sv, sv * 0.2)
                                acc = acc + lv * att_rows[c0 // 16][c0 % 16]
                                if ACT <= 16:
                                    saved.append((c0, xlv))
                            wv = jnp.exp(acc)
                            if ACT <= 16:
                                for c0, xlv in saved:
                                    cs = jnp.full((16,), h * ACT + c0,
                                                  jnp.int32)
                                    plsc.store_scatter(wb, [rowv, cs],
                                                       xlv * wv)
                            else:
                                for c0 in range(ACT):
                                    cg = jnp.full((16,), h * CH + c0,
                                                  jnp.int32)
                                    cs = jnp.full((16,), h * ACT + c0,
                                                  jnp.int32)
                                    xlv = plsc.load_gather(xb, [rowv, cg])
                                    plsc.store_scatter(wb, [rowv, cs],
                                                       xlv * wv)
                            plsc.store_scatter(
                                wb, [rowv, jnp.full((16,), DS + h, jnp.int32)],
                                wv)
                        return carry3

                    lax.fori_loop(0, K // 16, eg_body, 0)
                    pltpu.async_copy(wxlb.at[b % 2], acc_s.at[dstb.at[jl]],
                                     ssems[b % 2], add=True)
                    @pl.when(jl + RING < CHUNK_B)
                    def _():
                        issue_gather(jl + RING, b)
                return carry2

            lax.fori_loop(0, CHUNK_B // RING, group, 0)
            return carry

        lax.fori_loop(0, NCHUNK, chunk, 0)
        drain_scatter(0)
        drain_scatter(1)
        plsc.subcore_barrier()
        pltpu.sync_copy(acc_s.at[pl.ds(s * _ROWS_PER_TILE, _ROWS_PER_TILE)],
                        out_h.at[c, pl.ds(s * _ROWS_PER_TILE, _ROWS_PER_TILE)])

    return body


_K1 = 32
_EP1 = 331776
_K2 = 32
_EP2 = 331776
_sc_layer1 = _sc_layer(128, 8, 16, 144, _K1, _EP1, 54, RING=2)
_sc_layer2 = _sc_layer(48, 1, 40, 48, _K2, _EP2, 108, RING=2, TBL_SPMEM=True)


def _tc_mm2(x, Wa, Wb):
    n, f = x.shape
    d = Wa.shape[1]
    B = 1024

    def body(x_r, wa_r, wb_r, oa_r, ob_r):
        xb = x_r[...]
        oa_r[...] = jnp.dot(xb, wa_r[...], preferred_element_type=jnp.float32)
        ob_r[...] = jnp.dot(xb, wb_r[...], preferred_element_type=jnp.float32)

    return pl.pallas_call(
        body,
        grid=(n // B,),
        in_specs=[
            pl.BlockSpec((B, f), lambda i: (i, 0)),
            pl.BlockSpec((f, d), lambda i: (0, 0)),
            pl.BlockSpec((f, d), lambda i: (0, 0)),
        ],
        out_specs=[
            pl.BlockSpec((B, d), lambda i: (i, 0)),
            pl.BlockSpec((B, d), lambda i: (i, 0)),
        ],
        out_shape=[
            jax.ShapeDtypeStruct((n, d), jnp.float32),
            jax.ShapeDtypeStruct((n, d), jnp.float32),
        ],
    )(x, Wa, Wb)


def _tc_combine1_mm(p0, p1, b1, Wl2, Wr2):
    n = p0.shape[0]
    B = 1024
    d2 = Wl2.shape[1]

    def body(p0_r, p1_r, b1_r, wl_r, wr_r, oa_r, ob_r):
        p = p0_r[...] + p1_r[...]
        num = p[:, :128]
        den = p[:, 128:136]
        denb = jnp.broadcast_to(den.reshape(B, 8, 1), (B, 8, 16)).reshape(B, 128)
        h = jnp.maximum(num / (denb + 1e-16) + b1_r[...], 0.0)
        oa_r[...] = jnp.dot(h, wl_r[...], preferred_element_type=jnp.float32)
        ob_r[...] = jnp.dot(h, wr_r[...], preferred_element_type=jnp.float32)

    return pl.pallas_call(
        body,
        grid=(n // B,),
        in_specs=[
            pl.BlockSpec((B, 144), lambda i: (i, 0)),
            pl.BlockSpec((B, 144), lambda i: (i, 0)),
            pl.BlockSpec((1, 128), lambda i: (0, 0)),
            pl.BlockSpec((128, d2), lambda i: (0, 0)),
            pl.BlockSpec((128, d2), lambda i: (0, 0)),
        ],
        out_specs=[
            pl.BlockSpec((B, d2), lambda i: (i, 0)),
            pl.BlockSpec((B, d2), lambda i: (i, 0)),
        ],
        out_shape=[
            jax.ShapeDtypeStruct((n, d2), jnp.float32),
            jax.ShapeDtypeStruct((n, d2), jnp.float32),
        ],
    )(p0, p1, b1, Wl2, Wr2)


def _tc_combine2(q0, q1, b2):
    n = q0.shape[0]
    B = 1024

    def body(q0_r, q1_r, b2_r, o_r):
        q = q0_r[...] + q1_r[...]
        den = jnp.broadcast_to(q[:, 40:41], (B, 48))
        o_r[...] = q / (den + 1e-16) + b2_r[...]

    return pl.pallas_call(
        body,
        grid=(n // B,),
        in_specs=[
            pl.BlockSpec((B, 48), lambda i: (i, 0)),
            pl.BlockSpec((B, 48), lambda i: (i, 0)),
            pl.BlockSpec((1, 48), lambda i: (0, 0)),
        ],
        out_specs=pl.BlockSpec((B, 48), lambda i: (i, 0)),
        out_shape=jax.ShapeDtypeStruct((n, 48), jnp.float32),
    )(q0, q1, b2)


def _pad_edges(adj_t, ep):
    loops = jnp.arange(_N, dtype=jnp.int32)
    padi = jnp.full((ep - _E - _N,), _N, dtype=jnp.int32)
    src = jnp.concatenate([adj_t[0].astype(jnp.int32), loops, padi])
    dst = jnp.concatenate([adj_t[1].astype(jnp.int32), loops, padi])
    return src, dst


def kernel(x, adj_t, Wl1, Wr1, att1, b1, Wl2, Wr2, att2, b2):
    src1, dst1 = _pad_edges(adj_t, _EP1)
    gidx1 = jnp.concatenate(
        [src1.reshape(-1, _K1), dst1.reshape(-1, _K1) + _NP], axis=1)
    dstb1 = dst1.reshape(-1, _K1)
    src2, dst2 = _pad_edges(adj_t, _EP2)
    gidx2 = jnp.concatenate(
        [src2.reshape(-1, _K2), dst2.reshape(-1, _K2) + _NP], axis=1)
    dstb2 = dst2.reshape(-1, _K2)

    xp = jnp.pad(x, ((0, _NP - _N), (0, 0)))
    xl1, xr1 = _tc_mm2(xp, Wl1, Wr1)
    t1 = jnp.concatenate([xl1, xr1])

    att1f = att1.reshape(128)
    parts1 = _sc_layer1(t1, gidx1, dstb1, att1f)

    Wl2p = jnp.pad(Wl2, ((0, 0), (0, 8)))
    Wr2p = jnp.pad(Wr2, ((0, 0), (0, 8)))
    hl2, hr2 = _tc_combine1_mm(parts1[0], parts1[1], b1.reshape(1, 128),
                               Wl2p, Wr2p)
    t2 = jnp.concatenate([hl2, hr2])

    att2f = jnp.pad(att2.reshape(40), (0, 8))
    parts2 = _sc_layer2(t2, gidx2, dstb2, att2f)

    b2p = jnp.pad(b2, (0, 8)).reshape(1, 48)
    outp = _tc_combine2(parts2[0], parts2[1], b2p)
    return outp[:_N, :40]


# speedup vs baseline: 1.3680x; 1.0007x over previous
import functools

import jax
import jax.numpy as jnp
from jax import lax
from jax.experimental import pallas as pl
from jax.experimental.pallas import tpu as pltpu
from jax.experimental.pallas import tpu_sc as plsc

_N = 10000
_E = 320000
_NP = 10240
_EP = 331776
_ROWS_PER_TILE = _NP // 16

_mesh = plsc.VectorSubcoreMesh(core_axis_name="c", subcore_axis_name="s")


def _sc_layer(D, H, ACT, ROWLEN, K, EP, CHUNK_B, RING=2, TBL_SPMEM=False):
    CH = D // H
    DS = H * ACT
    NB = EP // (32 * K)
    NCHUNK = NB // CHUNK_B
    TROWS = (2 * _NP) // 16

    @functools.partial(
        pl.kernel,
        mesh=_mesh,
        compiler_params=pltpu.CompilerParams(
            needs_layout_passes=False, use_tc_tiling_on_sc=False),
        out_type=jax.ShapeDtypeStruct((2, _NP, ROWLEN), jnp.float32),
        scratch_types=(
            [pltpu.VMEM_SHARED((_NP, ROWLEN), jnp.float32)]
            + ([pltpu.VMEM_SHARED((2 * _NP, D), jnp.float32)]
               if TBL_SPMEM else [])
            + [
                pltpu.VMEM((CHUNK_B, 2 * K), jnp.int32),
                pltpu.VMEM((CHUNK_B, K), jnp.int32),
                pltpu.VMEM((RING, 2 * K, D), jnp.float32),
                pltpu.VMEM((2, K, ROWLEN), jnp.float32),
                pltpu.VMEM((D,), jnp.float32),
            ]
            + [pltpu.SemaphoreType.DMA] * (RING + 2)
        ),
    )
    def body(t_h, gidx_h, dst2_h, att_h, out_h, acc_s, *rest):
        if TBL_SPMEM:
            tbl_s = rest[0]
            rest = rest[1:]
        gidxb, dstb, xlrb, wxlb, att_v = rest[:5]
        gsems = rest[5:5 + RING]
        ssems = rest[5 + RING:5 + RING + 2]
        c = lax.axis_index("c")
        s = lax.axis_index("s")
        wid = s * 2 + c
        wbase = wid * NB

        pltpu.sync_copy(att_h, att_v)

        zv = jnp.zeros((16,), jnp.float32)

        def zb(r, carry):
            for b in range(2):
                for cc in range(ROWLEN // 16):
                    wxlb[b, r, pl.ds(cc * 16, 16)] = zv
            return carry

        lax.fori_loop(0, K, zb, 0)
        for t in range(_ROWS_PER_TILE // K):
            pltpu.async_copy(wxlb.at[0],
                             acc_s.at[pl.ds(s * _ROWS_PER_TILE + t * K, K)],
                             ssems[0])
        for t in range(_ROWS_PER_TILE // K):
            pltpu.make_async_copy(
                wxlb.at[0],
                acc_s.at[pl.ds(s * _ROWS_PER_TILE + t * K, K)],
                ssems[0]).wait()
        if TBL_SPMEM:
            pltpu.sync_copy(t_h.at[pl.ds(s * TROWS, TROWS)],
                            tbl_s.at[pl.ds(s * TROWS, TROWS)])
        plsc.subcore_barrier()

        t_src = tbl_s if TBL_SPMEM else t_h

        def drain_scatter(b):
            pltpu.make_async_copy(out_h.at[0, pl.ds(0, K)], wxlb.at[b],
                                  ssems[b]).wait()

        def load_chunk(cc):
            pltpu.sync_copy(gidx_h.at[pl.ds(wbase + cc * CHUNK_B, CHUNK_B)],
                            gidxb)
            pltpu.sync_copy(dst2_h.at[pl.ds(wbase + cc * CHUNK_B, CHUNK_B)],
                            dstb)

        def issue_gather(jl, b):
            pltpu.async_copy(t_src.at[gidxb.at[jl]], xlrb.at[b], gsems[b])

        load_chunk(0)
        for b in range(RING):
            issue_gather(b, b)

        def chunk(cc, carry):
            @pl.when(cc > 0)
            def _():
                drain_scatter(0)
                drain_scatter(1)
                load_chunk(cc)
                for b in range(RING):
                    issue_gather(b, b)

            def group(jj, carry2):
                for b in range(RING):
                    jl = jj * RING + b
                    pltpu.make_async_copy(t_src.at[gidxb.at[jl]],
                                          xlrb.at[b], gsems[b]).wait()
                    @pl.when(jl >= 2)
                    def _():
                        drain_scatter(b % 2)

                    def eg_body(eg, carry3):
                        rowv = eg * 16 + lax.iota(jnp.int32, 16)
                        xb = xlrb.at[b]
                        wb = wxlb.at[b % 2]
                        for h in range(H):
                            acc = jnp.zeros((16,), jnp.float32)
                            att_rows = [att_v[pl.ds(h * CH + k * 16, 16)]
                                        for k in range(-(-ACT // 16))]
                            saved = []
                            for c0 in range(ACT):
                                cg = jnp.full((16,), h * CH + c0, jnp.int32)
                                xlv = plsc.load_gather(xb, [rowv, cg])
                                xrv = plsc.load_gather(xb, [rowv + K, cg])
                                sv = xlv + xrv
                                lv = jnp.maximum(sv, sv * 0.2)
                                acc = acc + lv * att_rows[c0 // 16][c0 % 16]
                                if ACT <= 16:
                                    saved.append((c0, xlv))
                            wv = jnp.exp(acc)
                            if ACT <= 16:
                                for c0, xlv in saved:
                                    cs = jnp.full((16,), h * ACT + c0,
                                                  jnp.int32)
                                    plsc.store_scatter(wb, [rowv, cs],
                                                       xlv * wv)
                            else:
                                for c0 in range(ACT):
                                    cg = jnp.full((16,), h * CH + c0,
                                                  jnp.int32)
                                    cs = jnp.full((16,), h * ACT + c0,
                                                  jnp.int32)
                                    xlv = plsc.load_gather(xb, [rowv, cg])
                                    plsc.store_scatter(wb, [rowv, cs],
                                                       xlv * wv)
                            plsc.store_scatter(
                                wb, [rowv, jnp.full((16,), DS + h, jnp.int32)],
                                wv)
                        return carry3

                    lax.fori_loop(0, K // 16, eg_body, 0)
                    pltpu.async_copy(wxlb.at[b % 2], acc_s.at[dstb.at[jl]],
                                     ssems[b % 2], add=True)
                    @pl.when(jl + RING < CHUNK_B)
                    def _():
                        issue_gather(jl + RING, b)
                return carry2

            lax.fori_loop(0, CHUNK_B // RING, group, 0)
            return carry

        lax.fori_loop(0, NCHUNK, chunk, 0)
        drain_scatter(0)
        drain_scatter(1)
        plsc.subcore_barrier()
        pltpu.sync_copy(acc_s.at[pl.ds(s * _ROWS_PER_TILE, _ROWS_PER_TILE)],
                        out_h.at[c, pl.ds(s * _ROWS_PER_TILE, _ROWS_PER_TILE)])

    return body


_K1 = 32
_EP1 = 331776
_K2 = 32
_EP2 = 331776
_sc_layer1 = _sc_layer(128, 8, 16, 144, _K1, _EP1, 54, RING=2)
_sc_layer2 = _sc_layer(48, 1, 40, 48, _K2, _EP2, 108, RING=2, TBL_SPMEM=True)


def _tc_mm2(x, Wa, Wb):
    n, f = x.shape
    d = Wa.shape[1]
    B = 1024

    def body(x_r, wa_r, wb_r, oa_r, ob_r):
        xb = x_r[...]
        oa_r[...] = jnp.dot(xb, wa_r[...], preferred_element_type=jnp.float32)
        ob_r[...] = jnp.dot(xb, wb_r[...], preferred_element_type=jnp.float32)

    return pl.pallas_call(
        body,
        grid=(n // B,),
        in_specs=[
            pl.BlockSpec((B, f), lambda i: (i, 0)),
            pl.BlockSpec((f, d), lambda i: (0, 0)),
            pl.BlockSpec((f, d), lambda i: (0, 0)),
        ],
        out_specs=[
            pl.BlockSpec((B, d), lambda i: (i, 0)),
            pl.BlockSpec((B, d), lambda i: (i, 0)),
        ],
        out_shape=[
            jax.ShapeDtypeStruct((n, d), jnp.float32),
            jax.ShapeDtypeStruct((n, d), jnp.float32),
        ],
    )(x, Wa, Wb)


def _tc_combine1_mm(p0, p1, b1, Wl2, Wr2):
    n = p0.shape[0]
    B = 1024
    d2 = Wl2.shape[1]

    def body(p0_r, p1_r, b1_r, wl_r, wr_r, oa_r, ob_r):
        p = p0_r[...] + p1_r[...]
        num = p[:, :128]
        den = p[:, 128:136]
        denb = jnp.broadcast_to(den.reshape(B, 8, 1), (B, 8, 16)).reshape(B, 128)
        h = jnp.maximum(num / (denb + 1e-16) + b1_r[...], 0.0)
        oa_r[...] = jnp.dot(h, wl_r[...], preferred_element_type=jnp.float32)
        ob_r[...] = jnp.dot(h, wr_r[...], preferred_element_type=jnp.float32)

    return pl.pallas_call(
        body,
        grid=(n // B,),
        in_specs=[
            pl.BlockSpec((B, 144), lambda i: (i, 0)),
            pl.BlockSpec((B, 144), lambda i: (i, 0)),
            pl.BlockSpec((1, 128), lambda i: (0, 0)),
            pl.BlockSpec((128, d2), lambda i: (0, 0)),
            pl.BlockSpec((128, d2), lambda i: (0, 0)),
        ],
        out_specs=[
            pl.BlockSpec((B, d2), lambda i: (i, 0)),
            pl.BlockSpec((B, d2), lambda i: (i, 0)),
        ],
        out_shape=[
            jax.ShapeDtypeStruct((n, d2), jnp.float32),
            jax.ShapeDtypeStruct((n, d2), jnp.float32),
        ],
    )(p0, p1, b1, Wl2, Wr2)


def _tc_combine2(q0, q1, b2):
    n = q0.shape[0]
    B = 1024

    def body(q0_r, q1_r, b2_r, o_r):
        q = q0_r[...] + q1_r[...]
        den = jnp.broadcast_to(q[:, 40:41], (B, 48))
        o_r[...] = q / (den + 1e-16) + b2_r[...]

    return pl.pallas_call(
        body,
        grid=(n // B,),
        in_specs=[
            pl.BlockSpec((B, 48), lambda i: (i, 0)),
            pl.BlockSpec((B, 48), lambda i: (i, 0)),
            pl.BlockSpec((1, 48), lambda i: (0, 0)),
        ],
        out_specs=pl.BlockSpec((B, 48), lambda i: (i, 0)),
        out_shape=jax.ShapeDtypeStruct((n, 48), jnp.float32),
    )(q0, q1, b2)


def _pad_edges(adj_t, ep):
    loops = jnp.arange(_N, dtype=jnp.int32)
    padi = jnp.full((ep - _E - _N,), _N, dtype=jnp.int32)
    src = jnp.concatenate([adj_t[0].astype(jnp.int32), loops, padi])
    dst = jnp.concatenate([adj_t[1].astype(jnp.int32), loops, padi])
    return src, dst


def kernel(x, adj_t, Wl1, Wr1, att1, b1, Wl2, Wr2, att2, b2):
    src1, dst1 = _pad_edges(adj_t, _EP1)
    gidx1 = jnp.concatenate(
        [src1.reshape(-1, _K1), dst1.reshape(-1, _K1) + _NP], axis=1)
    dstb1 = dst1.reshape(-1, _K1)
    src2, dst2 = _pad_edges(adj_t, _EP2)
    gidx2 = jnp.concatenate(
        [src2.reshape(-1, _K2), dst2.reshape(-1, _K2) + _NP], axis=1)
    dstb2 = dst2.reshape(-1, _K2)

    xp = jnp.pad(x, ((0, _NP - _N), (0, 0)))
    xl1, xr1 = _tc_mm2(xp, Wl1, Wr1)
    t1 = jnp.concatenate([xl1, xr1])

    att1f = att1.reshape(128)
    parts1 = _sc_layer1(t1, gidx1, dstb1, att1f)

    Wl2p = jnp.pad(Wl2, ((0, 0), (0, 8)))
    Wr2p = jnp.pad(Wr2, ((0, 0), (0, 8)))
    hl2, hr2 = _tc_combine1_mm(parts1[0], parts1[1], b1.reshape(1, 128),
                               Wl2p, Wr2p)
    t2 = jnp.concatenate([hl2, hr2])

    att2f = jnp.pad(att2.reshape(40), (0, 8))
    parts2 = _sc_layer2(t2, gidx2, dstb2, att2f)

    b2p = jnp.pad(b2, (0, 8)).reshape(1, 48)
    outp = _tc_combine2(parts2[0], parts2[1], b2p)
    return outp[:_N, :40]
